# baseline (speedup 1.0000x reference)
import functools

import jax
import jax.numpy as jnp
from jax import lax
from jax.experimental import pallas as pl
from jax.experimental.pallas import tpu as pltpu
from jax.experimental.pallas import tpu_sc as plsc

B, K, C, NPOINT = 16, 16384, 256, 4096
L = 16
HALF = NPOINT // 2
CPW = C // 2
NBUF = 6
NOBUF = 2
UNROLL = 8
MAIN = (CPW // NBUF) * NBUF


def _sc_gather_kernel(xyzt_hbm, feat_hbm, idx_hbm, oxyzt_hbm, ofeat_hbm,
                      idx_v, oxyzt_v,
                      row0_v, row1_v, row2_v, row3_v, row4_v, row5_v,
                      orow0_v, orow1_v,
                      sem_xyz, sem_in, sem_out):
    rows = (row0_v, row1_v, row2_v, row3_v, row4_v, row5_v)
    orows = (orow0_v, orow1_v)

    cid = lax.axis_index("c")
    sid = lax.axis_index("s")
    wid = sid * 2 + cid
    b = wid // 2
    h = wid % 2
    c0 = h * CPW

    xyz_in = []
    for comp in range(3):
        cp = pltpu.make_async_copy(xyzt_hbm.at[b * 3 + comp], rows[comp],
                                   sem_xyz)
        cp.start()
        xyz_in.append(cp)
    for u in range(3, NBUF):
        pltpu.make_async_copy(feat_hbm.at[b, c0 + u], rows[u],
                              sem_in.at[u]).start()

    pltpu.sync_copy(idx_hbm.at[b], idx_v)

    for cp in xyz_in:
        cp.wait()

    def xyz_body(j, carry):
        for k in range(4):
            base = j * 4 * L + k * L
            idxv = idx_v[pl.ds(h * HALF + base, L)]
            for comp in range(3):
                vals = plsc.load_gather(rows[comp], [idxv])
                oxyzt_v[pl.ds(comp * HALF + base, L)] = vals
        return carry

    lax.fori_loop(0, HALF // (4 * L), xyz_body, 0)
    oxyz_out = []
    for comp in range(3):
        cp = pltpu.make_async_copy(
            oxyzt_v.at[pl.ds(comp * HALF, HALF)],
            oxyzt_hbm.at[b * 3 + comp, pl.ds(h * HALF, HALF)], sem_xyz)
        cp.start()
        oxyz_out.append(cp)

    for u in range(3):
        pltpu.make_async_copy(feat_hbm.at[b, c0 + u], rows[u],
                              sem_in.at[u]).start()

    def do_row(r, u, o, wait_out_cond, prefetch):
        c = c0 + r
        pltpu.make_async_copy(feat_hbm.at[b, c], rows[u],
                              sem_in.at[u]).wait()

        def _wait_out():
            pltpu.make_async_copy(orows[o], ofeat_hbm.at[b, c],
                                  sem_out.at[o]).wait()
        if wait_out_cond is None:
            _wait_out()
        else:
            pl.when(wait_out_cond)(_wait_out)

        rowref = rows[u]
        orowref = orows[o]

        def gat(i, carry2):
            for k in range(UNROLL):
                off = i * (UNROLL * L) + k * L
                idxv = idx_v[pl.ds(off, L)]
                orowref[pl.ds(off, L)] = plsc.load_gather(rowref, [idxv])
            return carry2

        lax.fori_loop(0, NPOINT // (UNROLL * L), gat, 0)

        if prefetch:
            @pl.when(r + NBUF < CPW)
            def _():
                pltpu.make_async_copy(feat_hbm.at[b, c + NBUF], rows[u],
                                      sem_in.at[u]).start()

        pltpu.make_async_copy(orows[o], ofeat_hbm.at[b, c],
                              sem_out.at[o]).start()

    def feat_round(g, carry):
        for u in range(NBUF):
            r = g * NBUF + u
            o = u % NOBUF
            do_row(r, u, o, (g > 0) if u < NOBUF else None, True)
        return carry

    lax.fori_loop(0, MAIN // NBUF, feat_round, 0)

    for r in range(MAIN, CPW):
        do_row(r, r % NBUF, r % NOBUF, None, False)

    for o in range(NOBUF):
        pltpu.make_async_copy(orows[o], ofeat_hbm.at[b, c0 + CPW - NOBUF + o],
                              sem_out.at[o]).wait()
    for cp in oxyz_out:
        cp.wait()


@jax.jit
def _sc_gather(xyz, features, sample_inds):
    mesh = plsc.VectorSubcoreMesh(core_axis_name="c", subcore_axis_name="s")
    kfn = functools.partial(
        pl.kernel,
        mesh=mesh,
        compiler_params=pltpu.CompilerParams(needs_layout_passes=False),
        out_type=[
            jax.ShapeDtypeStruct((B * 3, NPOINT), jnp.float32),
            jax.ShapeDtypeStruct((B, C, NPOINT), jnp.float32),
        ],
        scratch_types=[
            pltpu.VMEM((NPOINT,), jnp.int32),
            pltpu.VMEM((3 * HALF,), jnp.float32),
            pltpu.VMEM((K,), jnp.float32),
            pltpu.VMEM((K,), jnp.float32),
            pltpu.VMEM((K,), jnp.float32),
            pltpu.VMEM((K,), jnp.float32),
            pltpu.VMEM((K,), jnp.float32),
            pltpu.VMEM((K,), jnp.float32),
            pltpu.VMEM((NPOINT,), jnp.float32),
            pltpu.VMEM((NPOINT,), jnp.float32),
            pltpu.SemaphoreType.DMA,
            pltpu.SemaphoreType.DMA((NBUF,)),
            pltpu.SemaphoreType.DMA((NOBUF,)),
        ],
    )(_sc_gather_kernel)
    xyzt = jnp.swapaxes(xyz, 1, 2).reshape(B * 3, K)
    oxyzt, ofeat = kfn(xyzt, features, sample_inds)
    return jnp.swapaxes(oxyzt.reshape(B, 3, NPOINT), 1, 2), ofeat


def kernel(xyz, features, sample_inds):
    new_xyz, new_features = _sc_gather(xyz, features, sample_inds)
    return (new_xyz, new_features, sample_inds)


# speedup vs baseline: 1.0174x; 1.0174x over previous
import functools

import jax
import jax.numpy as jnp
from jax import lax
from jax.experimental import pallas as pl
from jax.experimental.pallas import tpu as pltpu
from jax.experimental.pallas import tpu_sc as plsc

B, K, C, NPOINT = 16, 16384, 256, 4096
L = 16
HALF = NPOINT // 2
CPW = C // 2
NBUF = 4
NOBUF = 2
UNROLL = 16


def _sc_gather_kernel(xyzt_hbm, feat_hbm, idx_hbm, oxyzt_hbm, ofeat_hbm,
                      idx_v, oxyzt_v,
                      row0_v, row1_v, row2_v, row3_v, orow0_v, orow1_v,
                      sem_xyz, sem_in, sem_out):
    rows = (row0_v, row1_v, row2_v, row3_v)
    orows = (orow0_v, orow1_v)

    cid = lax.axis_index("c")
    sid = lax.axis_index("s")
    wid = sid * 2 + cid
    b = wid // 2
    h = wid % 2
    c0 = h * CPW

    pltpu.sync_copy(idx_hbm.at[b], idx_v)

    xyz_in = []
    for comp in range(3):
        cp = pltpu.make_async_copy(xyzt_hbm.at[b * 3 + comp], rows[comp],
                                   sem_xyz)
        cp.start()
        xyz_in.append(cp)
    pltpu.make_async_copy(feat_hbm.at[b, c0 + 3], rows[3],
                          sem_in.at[3]).start()

    for cp in xyz_in:
        cp.wait()

    def xyz_body(j, carry):
        for k in range(4):
            base = j * 4 * L + k * L
            idxv = idx_v[pl.ds(h * HALF + base, L)]
            for comp in range(3):
                vals = plsc.load_gather(rows[comp], [idxv])
                oxyzt_v[pl.ds(comp * HALF + base, L)] = vals
        return carry

    lax.fori_loop(0, HALF // (4 * L), xyz_body, 0)
    oxyz_out = []
    for comp in range(3):
        cp = pltpu.make_async_copy(
            oxyzt_v.at[pl.ds(comp * HALF, HALF)],
            oxyzt_hbm.at[b * 3 + comp, pl.ds(h * HALF, HALF)], sem_xyz)
        cp.start()
        oxyz_out.append(cp)

    for u in range(3):
        pltpu.make_async_copy(feat_hbm.at[b, c0 + u], rows[u],
                              sem_in.at[u]).start()

    def feat_round(g, carry):
        for u in range(NBUF):
            r = g * NBUF + u
            c = c0 + r
            o = u % NOBUF
            pltpu.make_async_copy(feat_hbm.at[b, c], rows[u],
                                  sem_in.at[u]).wait()

            def _wait_out():
                pltpu.make_async_copy(orows[o], ofeat_hbm.at[b, c],
                                      sem_out.at[o]).wait()
            if u < NOBUF:
                pl.when(g > 0)(_wait_out)
            else:
                _wait_out()

            rowref = rows[u]
            orowref = orows[o]

            def gat(i, carry2):
                for k in range(UNROLL):
                    off = i * (UNROLL * L) + k * L
                    idxv = idx_v[pl.ds(off, L)]
                    orowref[pl.ds(off, L)] = plsc.load_gather(rowref, [idxv])
                return carry2

            lax.fori_loop(0, NPOINT // (UNROLL * L), gat, 0)

            @pl.when(r + NBUF < CPW)
            def _():
                pltpu.make_async_copy(feat_hbm.at[b, c + NBUF], rows[u],
                                      sem_in.at[u]).start()

            pltpu.make_async_copy(orows[o], ofeat_hbm.at[b, c],
                                  sem_out.at[o]).start()
        return carry

    lax.fori_loop(0, CPW // NBUF, feat_round, 0)

    for o in range(NOBUF):
        pltpu.make_async_copy(orows[o], ofeat_hbm.at[b, c0 + CPW - NOBUF + o],
                              sem_out.at[o]).wait()
    for cp in oxyz_out:
        cp.wait()


@jax.jit
def _sc_gather(xyz, features, sample_inds):
    mesh = plsc.VectorSubcoreMesh(core_axis_name="c", subcore_axis_name="s")
    kfn = functools.partial(
        pl.kernel,
        mesh=mesh,
        compiler_params=pltpu.CompilerParams(needs_layout_passes=False),
        out_type=[
            jax.ShapeDtypeStruct((B * 3, NPOINT), jnp.float32),
            jax.ShapeDtypeStruct((B, C, NPOINT), jnp.float32),
        ],
        scratch_types=[
            pltpu.VMEM((NPOINT,), jnp.int32),
            pltpu.VMEM((3 * HALF,), jnp.float32),
            pltpu.VMEM((K,), jnp.float32),
            pltpu.VMEM((K,), jnp.float32),
            pltpu.VMEM((K,), jnp.float32),
            pltpu.VMEM((K,), jnp.float32),
            pltpu.VMEM((NPOINT,), jnp.float32),
            pltpu.VMEM((NPOINT,), jnp.float32),
            pltpu.SemaphoreType.DMA,
            pltpu.SemaphoreType.DMA((NBUF,)),
            pltpu.SemaphoreType.DMA((NOBUF,)),
        ],
    )(_sc_gather_kernel)
    xyzt = jnp.swapaxes(xyz, 1, 2).reshape(B * 3, K)
    oxyzt, ofeat = kfn(xyzt, features, sample_inds)
    return jnp.swapaxes(oxyzt.reshape(B, 3, NPOINT), 1, 2), ofeat


def kernel(xyz, features, sample_inds):
    new_xyz, new_features = _sc_gather(xyz, features, sample_inds)
    return (new_xyz, new_features, sample_inds)
